# baseline (speedup 1.0000x reference)
import functools

import jax
import jax.numpy as jnp
from jax.experimental import pallas as pl
from jax.experimental.pallas import tpu as pltpu


def _cdiv(a, b):
    return (a + b - 1) // b


def _mlp_sigmoid_sum_kernel(x_ref, w1_ref, b1_ref, w2_ref, b2_ref, out_ref, *,
                            tm, m_total, masked):
    xb = x_ref[...].astype(jnp.bfloat16)
    h = jnp.dot(xb, w1_ref[...], preferred_element_type=jnp.float32)
    h = jnp.maximum(h + b1_ref[...], 0.0).astype(jnp.bfloat16)

    yt = jax.lax.dot_general(
        w2_ref[...], h,
        dimension_numbers=(((1,), (1,)), ((), ())),
        preferred_element_type=jnp.float32)
    sig = jax.nn.sigmoid(yt + b2_ref[0])

    if masked:
        i = pl.program_id(0)
        lane = jax.lax.broadcasted_iota(jnp.int32, sig.shape, 1)
        sig = jnp.where(i * tm + lane < m_total, sig, 0.0)

    partial = jnp.sum(sig) * 0.125
    out_ref[...] = jnp.full(out_ref.shape, partial, dtype=out_ref.dtype)


def _pick_tm(m):
    for tm in (4096, 2048, 1024, 512):
        if m % tm == 0:
            return tm, False
    return min(4096, _cdiv(m, 8) * 8), True


def kernel(x, w1, b1, w2, b2):
    d = x.shape[-1]
    x2d = x.reshape(-1, d).astype(jnp.float32)
    m, _ = x2d.shape
    h_dim = w1.shape[0]

    tm, masked = _pick_tm(m)
    m_pad = _cdiv(m, tm) * tm
    if m_pad != m:
        x2d = jnp.pad(x2d, ((0, m_pad - m), (0, 0)))
    num_blocks = m_pad // tm

    w1_bf = jnp.asarray(w1, jnp.float32).T.astype(jnp.bfloat16)
    b1_r = jnp.asarray(b1, jnp.float32).reshape(1, h_dim)
    w2_rep = jnp.broadcast_to(
        jnp.asarray(w2, jnp.float32).reshape(1, h_dim), (8, h_dim)
    ).astype(jnp.bfloat16)
    b2_s = jnp.asarray(b2, jnp.float32).reshape(1)

    body = functools.partial(_mlp_sigmoid_sum_kernel,
                             tm=tm, m_total=m, masked=masked)

    partials = pl.pallas_call(
        body,
        out_shape=jax.ShapeDtypeStruct((num_blocks, 1, 128), jnp.float32),
        grid=(num_blocks,),
        in_specs=[
            pl.BlockSpec((tm, d), lambda i: (i, 0)),
            pl.BlockSpec((d, h_dim), lambda i: (0, 0)),
            pl.BlockSpec((1, h_dim), lambda i: (0, 0)),
            pl.BlockSpec((8, h_dim), lambda i: (0, 0)),
            pl.BlockSpec(memory_space=pltpu.MemorySpace.SMEM),
        ],
        out_specs=pl.BlockSpec((1, 1, 128), lambda i: (i, 0, 0)),
        compiler_params=pltpu.CompilerParams(
            dimension_semantics=("parallel",),
        ),
    )(x2d, w1_bf, b1_r, w2_rep, b2_s)

    return 0.5 + 0.5 * jnp.sum(partials[:, 0, 0]) / m
